# Initial kernel scaffold; baseline (speedup 1.0000x reference)
import jax
import jax.numpy as jnp
from jax.experimental import pallas as pl


def kernel(user_embedding, item_embedding, all_h, all_t):
    raise NotImplementedError("write your pallas kernel here")


# speedup vs baseline: 2.3537x; 2.3537x over previous
import functools

import jax
import jax.numpy as jnp
from jax.experimental import pallas as pl
from jax.experimental.pallas import tpu as pltpu

_EMB = 64
_NF = 4
_SPLIT = _EMB // _NF
_N_LAYERS = 2
_N_ITERS = 2


def _mean3_body(a_ref, b_ref, c_ref, o_ref):
    o_ref[...] = (a_ref[...] + b_ref[...] + c_ref[...]) * (1.0 / 3.0)


def _mean3(a, b, c):
    n = a.shape[0]
    blk = 2000
    return pl.pallas_call(
        _mean3_body,
        out_shape=jax.ShapeDtypeStruct(a.shape, a.dtype),
        grid=(n // blk,),
        in_specs=[pl.BlockSpec((blk, _EMB), lambda i: (i, 0))] * 3,
        out_specs=pl.BlockSpec((blk, _EMB), lambda i: (i, 0)),
    )(a, b, c)


def _norm_f(x):
    n = x.shape[0]
    xr = x.reshape(n, _NF, _SPLIT)
    nrm = jnp.sqrt(jnp.sum(xr * xr, axis=2, keepdims=True))
    return (xr / jnp.maximum(nrm, 1e-12)).reshape(n, _EMB)


def kernel(user_embedding, item_embedding, all_h, all_t):
    n_users = user_embedding.shape[0]
    N = n_users + item_embedding.shape[0]
    E = all_h.shape[0]
    ego = jnp.concatenate([user_embedding, item_embedding], axis=0)
    A = jnp.ones((_NF, E), dtype=jnp.float32)
    layer_outs = [ego]
    for layer in range(_N_LAYERS):
        Tn = jnp.tanh(_norm_f(ego))
        ego_r = ego.reshape(N, _NF, _SPLIT)
        fe = None
        for it in range(_N_ITERS):
            scores = jax.nn.softmax(A, axis=0)
            deg = jax.ops.segment_sum(scores.T, all_h, num_segments=N)
            d_col = jax.lax.rsqrt(deg)
            y = (ego_r * d_col[:, :, None]).reshape(N, _EMB)
            msg = (jnp.repeat(scores.T, _SPLIT, axis=1)) * y[all_t]
            fe = jax.ops.segment_sum(msg, all_h, num_segments=N)
            fe = (fe.reshape(N, _NF, _SPLIT) * d_col[:, :, None]).reshape(N, _EMB)
            last_step = layer == _N_LAYERS - 1 and it == _N_ITERS - 1
            if not last_step:
                Fn = _norm_f(fe)
                prod = Fn[all_h] * Tn[all_t]
                A = A + jnp.sum(prod.reshape(E, _NF, _SPLIT), axis=2).T
        ego = fe
        layer_outs.append(ego)
    all_emb = _mean3(*layer_outs)
    return all_emb[:n_users], all_emb[n_users:]


# speedup vs baseline: 4.3623x; 1.8534x over previous
import functools

import jax
import jax.numpy as jnp
from jax import lax
from jax.experimental import pallas as pl
from jax.experimental.pallas import tpu as pltpu
from jax.experimental.pallas import tpu_sc as plsc

_EMB = 64
_NF = 4
_SPLIT = _EMB // _NF
_N_LAYERS = 2
_N_ITERS = 2

_NSC = 2
_NT = 16

_MSG_C = 256
_ATT_C = 512
_DEG_C = 1024


def _mds(start, size, mult):
    return pl.ds(pl.multiple_of(start, mult), size)


def _pad_to(x, n, axis=0):
    pad = [(0, 0)] * x.ndim
    pad[axis] = (0, n - x.shape[axis])
    return jnp.pad(x, pad)


def _mean3_body(a_ref, b_ref, c_ref, o_ref):
    o_ref[...] = (a_ref[...] + b_ref[...] + c_ref[...]) * (1.0 / 3.0)


def _mean3(a, b, c):
    n = a.shape[0]
    blk = n // 16
    return pl.pallas_call(
        _mean3_body,
        out_shape=jax.ShapeDtypeStruct(a.shape, a.dtype),
        grid=(16,),
        in_specs=[pl.BlockSpec((blk, _EMB), lambda i: (i, 0))] * 3,
        out_specs=pl.BlockSpec((blk, _EMB), lambda i: (i, 0)),
    )(a, b, c)


def _norm_f(x):
    n = x.shape[0]
    xr = x.reshape(n, _NF, _SPLIT)
    nrm = jnp.sqrt(jnp.sum(xr * xr, axis=2, keepdims=True))
    return (xr / jnp.maximum(nrm, 1e-12)).reshape(n, _EMB)


def _make_mesh():
    return plsc.VectorSubcoreMesh(core_axis_name="c", subcore_axis_name="s")


_SC_PARAMS = pltpu.CompilerParams(use_tc_tiling_on_sc=False)


def _make_deg_kernel(n_pad, e_pad):
    rows_pt = n_pad // _NT
    edges_pw = e_pad // (_NSC * _NT)
    chunks = edges_pw // _DEG_C

    @functools.partial(
        pl.kernel,
        out_type=jax.ShapeDtypeStruct((_NSC, n_pad, 4), jnp.float32),
        mesh=_make_mesh(),
        compiler_params=_SC_PARAMS,
        scratch_types=[
            pltpu.VMEM_SHARED((n_pad, 4), jnp.float32),
            pltpu.VMEM((8, 128), jnp.int32),
            pltpu.VMEM((_DEG_C, 4), jnp.float32),
            pltpu.SemaphoreType.DMA,
        ],
    )
    def deg_kernel(sv, h2, z4, dout, acc, idx_h, vals, sem):
        c = lax.axis_index("c")
        s = lax.axis_index("s")
        sl = _mds(s * rows_pt, rows_pt, 8)
        pltpu.sync_copy(z4.at[sl], acc.at[sl])
        plsc.subcore_barrier()
        base = (c * _NT + s) * edges_pw

        def chunk(i, carry):
            eb = base + i * _DEG_C
            pltpu.sync_copy(h2.at[_mds(eb // 128, _DEG_C // 128, 8)], idx_h)
            pltpu.async_copy(sv.at[_mds(eb, _DEG_C, 128)], vals, sem).wait()
            for j in range(_DEG_C // 128):
                pltpu.sync_copy(vals.at[pl.ds(j * 128, 128)],
                                acc.at[idx_h.at[j]], add=True)
            return carry

        lax.fori_loop(0, chunks, chunk, 0)
        plsc.subcore_barrier()
        pltpu.sync_copy(acc.at[sl], dout.at[c, sl])

    return deg_kernel


def _make_msg_kernel(n_pad, e_pad):
    rows_pt = n_pad // _NT
    edges_pt = e_pad // _NT
    chunks = edges_pt // _MSG_C

    @functools.partial(
        pl.kernel,
        out_type=jax.ShapeDtypeStruct((_NSC, n_pad, 32), jnp.float32),
        mesh=_make_mesh(),
        compiler_params=_SC_PARAMS,
        scratch_types=[
            pltpu.VMEM_SHARED((n_pad, 32), jnp.float32),
            pltpu.VMEM((_MSG_C // 128, 128), jnp.int32),
            pltpu.VMEM((_MSG_C // 128, 128), jnp.int32),
            pltpu.VMEM((_MSG_C, 32), jnp.float32),
            pltpu.VMEM((_MSG_C, 32), jnp.float32),
            pltpu.SemaphoreType.DMA,
        ],
    )
    def msg_kernel(y3, w3, h2, t2, z32, fe3,
                   acc, idx_h, idx_t, rows, wbuf, sem):
        c = lax.axis_index("c")
        s = lax.axis_index("s")
        sl = _mds(s * rows_pt, rows_pt, 8)
        pltpu.sync_copy(z32.at[sl], acc.at[sl])
        plsc.subcore_barrier()
        base = s * edges_pt

        def chunk(i, carry):
            eb = base + i * _MSG_C
            rb = eb // 128
            pltpu.sync_copy(t2.at[_mds(rb, _MSG_C // 128, 2)], idx_t)
            pltpu.sync_copy(h2.at[_mds(rb, _MSG_C // 128, 2)], idx_h)
            cps = [pltpu.async_copy(w3.at[c].at[_mds(eb, _MSG_C, 128)],
                                    wbuf, sem)]
            for j in range(_MSG_C // 128):
                cps.append(pltpu.async_copy(
                    y3.at[c].at[idx_t.at[j]],
                    rows.at[pl.ds(j * 128, 128)], sem))
            for cp in cps:
                cp.wait()

            def mulrow(r, cr):
                rows[r, pl.ds(0, 16)] = rows[r, pl.ds(0, 16)] * wbuf[r, pl.ds(0, 16)]
                rows[r, pl.ds(16, 16)] = rows[r, pl.ds(16, 16)] * wbuf[r, pl.ds(16, 16)]
                return cr

            lax.fori_loop(0, _MSG_C, mulrow, 0)
            for j in range(_MSG_C // 128):
                pltpu.sync_copy(rows.at[pl.ds(j * 128, 128)],
                                acc.at[idx_h.at[j]], add=True)
            return carry

        lax.fori_loop(0, chunks, chunk, 0)
        plsc.subcore_barrier()
        pltpu.sync_copy(acc.at[sl], fe3.at[c, sl])

    return msg_kernel


def _make_att_kernel(n_pad, e_pad):
    edges_pw = e_pad // (_NSC * _NT)
    chunks = edges_pw // _ATT_C

    @functools.partial(
        pl.kernel,
        out_type=jax.ShapeDtypeStruct((e_pad, _EMB), jnp.float32),
        mesh=_make_mesh(),
        compiler_params=_SC_PARAMS,
        scratch_types=[
            pltpu.VMEM((4, 128), jnp.int32),
            pltpu.VMEM((4, 128), jnp.int32),
            pltpu.VMEM((_ATT_C, _EMB), jnp.float32),
            pltpu.VMEM((_ATT_C, _EMB), jnp.float32),
            pltpu.SemaphoreType.DMA,
        ],
    )
    def att_kernel(fn, tn, h2, t2, out, idx_h, idx_t, ra, rb, sem):
        c = lax.axis_index("c")
        s = lax.axis_index("s")
        base = (c * _NT + s) * edges_pw

        def chunk(i, carry):
            eb = base + i * _ATT_C
            rbase = eb // 128
            pltpu.sync_copy(h2.at[_mds(rbase, _ATT_C // 128, 4)], idx_h)
            pltpu.sync_copy(t2.at[_mds(rbase, _ATT_C // 128, 4)], idx_t)
            cps = []
            for j in range(_ATT_C // 128):
                cps.append(pltpu.async_copy(
                    fn.at[idx_h.at[j]], ra.at[pl.ds(j * 128, 128)], sem))
                cps.append(pltpu.async_copy(
                    tn.at[idx_t.at[j]], rb.at[pl.ds(j * 128, 128)], sem))
            for cp in cps:
                cp.wait()

            def mulrow(r, cr):
                for q in range(_NF):
                    ra[r, pl.ds(q * 16, 16)] = (
                        ra[r, pl.ds(q * 16, 16)] * rb[r, pl.ds(q * 16, 16)])
                return cr

            lax.fori_loop(0, _ATT_C, mulrow, 0)
            pltpu.sync_copy(ra, out.at[_mds(eb, _ATT_C, 128)])
            return carry

        lax.fori_loop(0, chunks, chunk, 0)

    return att_kernel


def kernel(user_embedding, item_embedding, all_h, all_t):
    n_users = user_embedding.shape[0]
    N = n_users + item_embedding.shape[0]
    E = all_h.shape[0]
    n_pad = ((N + 1 + _NT * 8 - 1) // (_NT * 8)) * (_NT * 8)
    epc = _NSC * _NT * _MSG_C
    e_pad = ((E + epc - 1) // epc) * epc

    deg_k = _make_deg_kernel(n_pad, e_pad)
    msg_k = _make_msg_kernel(n_pad, e_pad)
    att_k = _make_att_kernel(n_pad, e_pad)

    ego = jnp.concatenate([user_embedding, item_embedding], axis=0)
    ego = _pad_to(ego, n_pad)
    h_p = _pad_to(all_h, e_pad).at[E:].set(N)
    t_p = _pad_to(all_t, e_pad).at[E:].set(N)
    h2 = h_p.reshape(e_pad // 128, 128)
    t2 = t_p.reshape(e_pad // 128, 128)
    z4 = jnp.zeros((n_pad, 4), jnp.float32)
    z32 = jnp.zeros((n_pad, 32), jnp.float32)

    A = jnp.ones((_NF, e_pad), dtype=jnp.float32)
    layer_outs = [ego]
    for layer in range(_N_LAYERS):
        Tn = jnp.tanh(_norm_f(ego))
        ego_r = ego.reshape(n_pad, _NF, _SPLIT)
        fe = None
        for it in range(_N_ITERS):
            scores = jax.nn.softmax(A, axis=0)
            sT = scores.T
            deg = deg_k(sT, h2, z4).sum(axis=0)
            d_col = lax.rsqrt(jnp.maximum(deg, 1e-30))
            y = (ego_r * d_col[:, :, None]).reshape(n_pad, _EMB)
            y3 = y.reshape(n_pad, _NSC, 32).transpose(1, 0, 2)
            wexp = jnp.repeat(sT, _SPLIT, axis=1)
            w3 = wexp.reshape(e_pad, _NSC, 32).transpose(1, 0, 2)
            fe3 = msg_k(y3, w3, h2, t2, z32)
            fe = fe3.transpose(1, 0, 2).reshape(n_pad, _EMB)
            fe = (fe.reshape(n_pad, _NF, _SPLIT) * d_col[:, :, None]
                  ).reshape(n_pad, _EMB)
            last_step = layer == _N_LAYERS - 1 and it == _N_ITERS - 1
            if not last_step:
                Fn = _norm_f(fe)
                P = att_k(Fn, Tn, h2, t2)
                A = A + jnp.sum(P.reshape(e_pad, _NF, _SPLIT), axis=2).T
        ego = fe
        layer_outs.append(ego)
    all_emb = _mean3(*layer_outs)
    return all_emb[:n_users], all_emb[n_users:N]
